# baseline (speedup 1.0000x reference)
import math

import jax
import jax.numpy as jnp
from jax.experimental import pallas as pl
from jax.experimental.pallas import tpu as pltpu

_N_Q = 6
_N_XI = 4
_NS = 8
_H = 32
_SR = _NS * _H
_XR = _NS * _N_XI

_ROW_WK, _ROW_PHASE, _ROW_B3 = 0, 8, 16
_ROW_W1, _ROW_W2 = 24, 56
_ROW_B1, _ROW_B2, _ROW_W3 = 88, 120, 152

_TILE = 4096
_CHUNK = 2048

_FLOPS_PER_CFG = 2 * (_XR * _N_Q + _SR * _XR + _SR * _SR + 3 * _NS * _SR)
_TRANS_PER_CFG = _XR + 2 * _SR

_INV_PI = 0.31830988618379067
_PI_HI = 3.140625
_PI_LO = 9.676535897932e-4
_C9 = 2.7557319e-6
_C7 = -1.9841270e-4
_C5 = 8.3333333e-3
_C3 = -0.16666667


def _fast_sin(x):
    kf = jnp.floor(x * _INV_PI + 0.5)
    r = x - kf * _PI_HI
    r = r - kf * _PI_LO
    r2 = r * r
    p = _C9 * r2 + _C7
    p = p * r2 + _C5
    p = p * r2 + _C3
    s = r * (p * r2 + 1.0)
    sbit = jax.lax.shift_left(jax.lax.bitwise_and(kf.astype(jnp.int32), 1), 31)
    bits = jax.lax.bitwise_xor(jax.lax.bitcast_convert_type(s, jnp.int32), sbit)
    return jax.lax.bitcast_convert_type(bits, jnp.float32)


def _hi_split(x):
    bits = jax.lax.bitcast_convert_type(x, jnp.int32)
    hi_bits = jax.lax.bitwise_and(bits, jnp.int32(-65536))
    return jax.lax.bitcast_convert_type(hi_bits, jnp.float32)


def _fwd_kernel(qT_ref, wk32_ref, ph32_ref, w1bd_ref, w2bd_ref,
                w3hi_ref, w3lo_ref, b1_ref, b2_ref, b3_ref, u_ref):
    for c in range(_TILE // _CHUNK):
        lo = c * _CHUNK
        z32 = jnp.dot(wk32_ref[...], qT_ref[:, lo:lo + _CHUNK],
                      preferred_element_type=jnp.float32)
        xi = _fast_sin(z32 + ph32_ref[...])
        h1 = jnp.tanh(jnp.dot(w1bd_ref[...], xi,
                              preferred_element_type=jnp.float32) + b1_ref[...])
        h2 = jnp.tanh(jnp.dot(w2bd_ref[...], h1,
                              preferred_element_type=jnp.float32) + b2_ref[...])
        h2hi = _hi_split(h2)
        h2lo = h2 - h2hi
        u = (jnp.dot(w3hi_ref[...], h2hi, preferred_element_type=jnp.float32)
             + jnp.dot(w3hi_ref[...], h2lo, preferred_element_type=jnp.float32)
             + jnp.dot(w3lo_ref[...], h2hi, preferred_element_type=jnp.float32))
        u_ref[:, lo:lo + _CHUNK] = u + b3_ref[0, 0]


@jax.jit
def kernel(Q, slab):
    wkT = slab[_ROW_WK:_ROW_WK + _N_XI, 0:_N_Q]
    phaseT = slab[_ROW_PHASE:_ROW_PHASE + _N_XI, 0:_NS]
    b3 = slab[_ROW_B3:_ROW_B3 + 1, 0:1]
    w1T = slab[_ROW_W1:_ROW_W1 + _H, 0:_N_XI]
    w2T = slab[_ROW_W2:_ROW_W2 + _H, 0:_H]
    b1T = slab[_ROW_B1:_ROW_B1 + _H, 0:1]
    b2T = slab[_ROW_B2:_ROW_B2 + _H, 0:1]
    w3c = slab[_ROW_W3:_ROW_W3 + _H, 0:1]

    wk32 = jnp.tile(wkT, (_NS, 1))
    ph32 = phaseT.T.reshape(_XR, 1)

    eye = jnp.eye(_NS, dtype=jnp.float32)
    w1bd = jnp.kron(eye, w1T)
    w2bd = jnp.kron(eye, w2T)
    w3bd = jnp.kron(eye, w3c.T)
    w3hi = _hi_split(w3bd)
    w3lo = w3bd - w3hi
    b1r = jnp.tile(b1T, (_NS, 1))
    b2r = jnp.tile(b2T, (_NS, 1))

    B = Q.shape[0]
    b_pad = ((B + _TILE - 1) // _TILE) * _TILE
    grid = b_pad // _TILE
    if b_pad == B:
        Qp = Q.astype(jnp.float32)
    else:
        Qp = jnp.zeros((b_pad, _N_Q), jnp.float32).at[:B].set(
            Q.astype(jnp.float32))
    qT = Qp.T

    whole = lambda shp: pl.BlockSpec(shp, lambda i: (0, 0))
    out = pl.pallas_call(
        _fwd_kernel,
        out_shape=jax.ShapeDtypeStruct((_NS, b_pad), jnp.float32),
        grid=(grid,),
        in_specs=[
            pl.BlockSpec((_N_Q, _TILE), lambda i: (0, i)),
            whole((_XR, _N_Q)), whole((_XR, 1)),
            whole((_SR, _XR)), whole((_SR, _SR)),
            whole((_NS, _SR)), whole((_NS, _SR)),
            whole((_SR, 1)), whole((_SR, 1)), whole((1, 1)),
        ],
        out_specs=pl.BlockSpec((_NS, _TILE), lambda i: (0, i)),
        compiler_params=pltpu.CompilerParams(
            dimension_semantics=("parallel",)),
        cost_estimate=pl.CostEstimate(
            flops=_FLOPS_PER_CFG * b_pad,
            transcendentals=_TRANS_PER_CFG * b_pad,
            bytes_accessed=4 * (_N_Q + _NS) * b_pad + 4 * _SR * _SR),
    )(qT, wk32, ph32, w1bd, w2bd, w3hi, w3lo, b1r, b2r, b3)
    return out[:, :B].T


# speedup vs baseline: 1.2295x; 1.2295x over previous
import math

import jax
import jax.numpy as jnp
from jax.experimental import pallas as pl
from jax.experimental.pallas import tpu as pltpu

_N_Q = 6
_N_XI = 4
_NS = 8
_H = 32
_SR = _NS * _H
_XR = _NS * _N_XI

_ROW_WK, _ROW_PHASE, _ROW_B3 = 0, 8, 16
_ROW_W1, _ROW_W2 = 24, 56
_ROW_B1, _ROW_B2, _ROW_W3 = 88, 120, 152

_TILE = 16384
_CHUNK = 8192

_FLOPS_PER_CFG = 2 * (_XR * _N_Q + _SR * _XR + _SR * _SR + 3 * _NS * _SR)
_TRANS_PER_CFG = _XR + 2 * _SR

_INV_PI = 0.31830988618379067
_PI_HI = 3.140625
_PI_LO = 9.676535897932e-4
_C9 = 2.7557319e-6
_C7 = -1.9841270e-4
_C5 = 8.3333333e-3
_C3 = -0.16666667


def _fast_sin(x):
    kf = jnp.floor(x * _INV_PI + 0.5)
    r = x - kf * _PI_HI
    r = r - kf * _PI_LO
    r2 = r * r
    p = _C9 * r2 + _C7
    p = p * r2 + _C5
    p = p * r2 + _C3
    s = r * (p * r2 + 1.0)
    sbit = jax.lax.shift_left(jax.lax.bitwise_and(kf.astype(jnp.int32), 1), 31)
    bits = jax.lax.bitwise_xor(jax.lax.bitcast_convert_type(s, jnp.int32), sbit)
    return jax.lax.bitcast_convert_type(bits, jnp.float32)


def _hi_split(x):
    bits = jax.lax.bitcast_convert_type(x, jnp.int32)
    hi_bits = jax.lax.bitwise_and(bits, jnp.int32(-65536))
    return jax.lax.bitcast_convert_type(hi_bits, jnp.float32)


def _fwd_kernel(qT_ref, wk2_ref, offs_ref, cph_ref, sph_ref, w1bd_ref, w2bd_ref,
                w3hi_ref, w3lo_ref, b1_ref, b2_ref, b3_ref, u_ref):
    for c in range(_TILE // _CHUNK):
        lo = c * _CHUNK
        z8 = jnp.dot(wk2_ref[...], qT_ref[:, lo:lo + _CHUNK],
                     preferred_element_type=jnp.float32)
        sc = _fast_sin(z8 + offs_ref[...])
        sin32 = jnp.tile(sc[0:_N_XI], (_NS, 1))
        cos32 = jnp.tile(sc[_N_XI:2 * _N_XI], (_NS, 1))
        xi = sin32 * cph_ref[...] + cos32 * sph_ref[...]
        h1 = jnp.tanh(jnp.dot(w1bd_ref[...], xi,
                              preferred_element_type=jnp.float32) + b1_ref[...])
        h2 = jnp.tanh(jnp.dot(w2bd_ref[...], h1,
                              preferred_element_type=jnp.float32) + b2_ref[...])
        h2hi = _hi_split(h2)
        h2lo = h2 - h2hi
        u = (jnp.dot(w3hi_ref[...], h2hi, preferred_element_type=jnp.float32)
             + jnp.dot(w3hi_ref[...], h2lo, preferred_element_type=jnp.float32)
             + jnp.dot(w3lo_ref[...], h2hi, preferred_element_type=jnp.float32))
        u_ref[:, lo:lo + _CHUNK] = u + b3_ref[0, 0]


@jax.jit
def kernel(Q, slab):
    wkT = slab[_ROW_WK:_ROW_WK + _N_XI, 0:_N_Q]
    phaseT = slab[_ROW_PHASE:_ROW_PHASE + _N_XI, 0:_NS]
    b3 = slab[_ROW_B3:_ROW_B3 + 1, 0:1]
    w1T = slab[_ROW_W1:_ROW_W1 + _H, 0:_N_XI]
    w2T = slab[_ROW_W2:_ROW_W2 + _H, 0:_H]
    b1T = slab[_ROW_B1:_ROW_B1 + _H, 0:1]
    b2T = slab[_ROW_B2:_ROW_B2 + _H, 0:1]
    w3c = slab[_ROW_W3:_ROW_W3 + _H, 0:1]

    wk2 = jnp.concatenate([wkT, wkT], axis=0)
    offs = jnp.concatenate([jnp.zeros((_N_XI, 1), jnp.float32),
                            jnp.full((_N_XI, 1), 0.5 * math.pi, jnp.float32)],
                           axis=0)
    cph32 = jnp.cos(phaseT).T.reshape(_XR, 1)
    sph32 = jnp.sin(phaseT).T.reshape(_XR, 1)

    eye = jnp.eye(_NS, dtype=jnp.float32)
    w1bd = jnp.kron(eye, w1T).astype(jnp.bfloat16)
    w2bd = jnp.kron(eye, w2T).astype(jnp.bfloat16)
    w3bd = jnp.kron(eye, w3c.T)
    w3hi_f = _hi_split(w3bd)
    w3hi = w3hi_f.astype(jnp.bfloat16)
    w3lo = (w3bd - w3hi_f).astype(jnp.bfloat16)
    b1r = jnp.tile(b1T, (_NS, 1))
    b2r = jnp.tile(b2T, (_NS, 1))

    B = Q.shape[0]
    b_pad = ((B + _TILE - 1) // _TILE) * _TILE
    grid = b_pad // _TILE
    if b_pad == B:
        Qp = Q.astype(jnp.float32)
    else:
        Qp = jnp.zeros((b_pad, _N_Q), jnp.float32).at[:B].set(
            Q.astype(jnp.float32))
    qT = Qp.T

    whole = lambda shp: pl.BlockSpec(shp, lambda i: (0, 0))
    out = pl.pallas_call(
        _fwd_kernel,
        out_shape=jax.ShapeDtypeStruct((_NS, b_pad), jnp.float32),
        grid=(grid,),
        in_specs=[
            pl.BlockSpec((_N_Q, _TILE), lambda i: (0, i)),
            whole((2 * _N_XI, _N_Q)), whole((2 * _N_XI, 1)),
            whole((_XR, 1)), whole((_XR, 1)),
            whole((_SR, _XR)), whole((_SR, _SR)),
            whole((_NS, _SR)), whole((_NS, _SR)),
            whole((_SR, 1)), whole((_SR, 1)), whole((1, 1)),
        ],
        out_specs=pl.BlockSpec((_NS, _TILE), lambda i: (0, i)),
        compiler_params=pltpu.CompilerParams(
            dimension_semantics=("parallel",)),
        cost_estimate=pl.CostEstimate(
            flops=_FLOPS_PER_CFG * b_pad,
            transcendentals=_TRANS_PER_CFG * b_pad,
            bytes_accessed=4 * (_N_Q + _NS) * b_pad + 4 * _SR * _SR),
    )(qT, wk2, offs, cph32, sph32, w1bd, w2bd, w3hi, w3lo, b1r, b2r, b3)
    return out[:, :B].T


# speedup vs baseline: 1.2377x; 1.0067x over previous
import math

import jax
import jax.numpy as jnp
from jax.experimental import pallas as pl
from jax.experimental.pallas import tpu as pltpu

_N_Q = 6
_N_XI = 4
_NS = 8
_H = 32
_SR = _NS * _H
_XR = _NS * _N_XI

_ROW_WK, _ROW_PHASE, _ROW_B3 = 0, 8, 16
_ROW_W1, _ROW_W2 = 24, 56
_ROW_B1, _ROW_B2, _ROW_W3 = 88, 120, 152

_TILE = 16384
_CHUNK = 8192

_FLOPS_PER_CFG = 2 * (_XR * _N_Q + _SR * _XR + _SR * _SR + 3 * _NS * _SR)
_TRANS_PER_CFG = _XR + 2 * _SR

_INV_PI = 0.31830988618379067
_PI_HI = 3.140625
_PI_LO = 9.676535897932e-4
_C9 = 2.7557319e-6
_C7 = -1.9841270e-4
_C5 = 8.3333333e-3
_C3 = -0.16666667


def _fast_sin(x):
    kf = jnp.floor(x * _INV_PI + 0.5)
    r = x - kf * _PI_HI
    r = r - kf * _PI_LO
    r2 = r * r
    p = _C9 * r2 + _C7
    p = p * r2 + _C5
    p = p * r2 + _C3
    s = r * (p * r2 + 1.0)
    sbit = jax.lax.shift_left(jax.lax.bitwise_and(kf.astype(jnp.int32), 1), 31)
    bits = jax.lax.bitwise_xor(jax.lax.bitcast_convert_type(s, jnp.int32), sbit)
    return jax.lax.bitcast_convert_type(bits, jnp.float32)


def _hi_split(x):
    bits = jax.lax.bitcast_convert_type(x, jnp.int32)
    hi_bits = jax.lax.bitwise_and(bits, jnp.int32(-65536))
    return jax.lax.bitcast_convert_type(hi_bits, jnp.float32)


def _fwd_kernel(qT_ref, wk2_ref, offs_ref, cph_ref, sph_ref, w14_ref, w24_ref,
                w3hi_ref, w3lo_ref, b1_ref, b2_ref, b3_ref, u_ref):
    for c in range(_TILE // _CHUNK):
        lo = c * _CHUNK
        z8 = jnp.dot(wk2_ref[...], qT_ref[:, lo:lo + _CHUNK],
                     preferred_element_type=jnp.float32)
        sc = _fast_sin(z8 + offs_ref[...])
        sin16 = jnp.tile(sc[0:_N_XI], (_NS // 2, 1))
        cos16 = jnp.tile(sc[_N_XI:2 * _N_XI], (_NS // 2, 1))
        b3s = b3_ref[0, 0]
        for h in range(2):
            r0 = h * 16
            xi = (sin16 * cph_ref[r0:r0 + 16] +
                  cos16 * sph_ref[r0:r0 + 16])
            h1 = jnp.tanh(jnp.dot(w14_ref[...], xi,
                                  preferred_element_type=jnp.float32) + b1_ref[...])
            h2 = jnp.tanh(jnp.dot(w24_ref[...], h1,
                                  preferred_element_type=jnp.float32) + b2_ref[...])
            h2hi = _hi_split(h2)
            h2lo = h2 - h2hi
            u = (jnp.dot(w3hi_ref[...], h2hi, preferred_element_type=jnp.float32)
                 + jnp.dot(w3hi_ref[...], h2lo, preferred_element_type=jnp.float32)
                 + jnp.dot(w3lo_ref[...], h2hi, preferred_element_type=jnp.float32))
            u_ref[h * 4:h * 4 + 4, lo:lo + _CHUNK] = u + b3s


@jax.jit
def kernel(Q, slab):
    wkT = slab[_ROW_WK:_ROW_WK + _N_XI, 0:_N_Q]
    phaseT = slab[_ROW_PHASE:_ROW_PHASE + _N_XI, 0:_NS]
    b3 = slab[_ROW_B3:_ROW_B3 + 1, 0:1]
    w1T = slab[_ROW_W1:_ROW_W1 + _H, 0:_N_XI]
    w2T = slab[_ROW_W2:_ROW_W2 + _H, 0:_H]
    b1T = slab[_ROW_B1:_ROW_B1 + _H, 0:1]
    b2T = slab[_ROW_B2:_ROW_B2 + _H, 0:1]
    w3c = slab[_ROW_W3:_ROW_W3 + _H, 0:1]

    wk2 = jnp.concatenate([wkT, wkT], axis=0)
    offs = jnp.concatenate([jnp.zeros((_N_XI, 1), jnp.float32),
                            jnp.full((_N_XI, 1), 0.5 * math.pi, jnp.float32)],
                           axis=0)
    cph32 = jnp.cos(phaseT).T.reshape(_XR, 1)
    sph32 = jnp.sin(phaseT).T.reshape(_XR, 1)

    eye4 = jnp.eye(_NS // 2, dtype=jnp.float32)
    w14 = jnp.kron(eye4, w1T).astype(jnp.bfloat16)
    w24 = jnp.kron(eye4, w2T).astype(jnp.bfloat16)
    w34 = jnp.kron(eye4, w3c.T)
    w3hi_f = _hi_split(w34)
    w3hi = w3hi_f.astype(jnp.bfloat16)
    w3lo = (w34 - w3hi_f).astype(jnp.bfloat16)
    b1r = jnp.tile(b1T, (_NS // 2, 1))
    b2r = jnp.tile(b2T, (_NS // 2, 1))

    B = Q.shape[0]
    b_pad = ((B + _TILE - 1) // _TILE) * _TILE
    grid = b_pad // _TILE
    if b_pad == B:
        Qp = Q.astype(jnp.float32)
    else:
        Qp = jnp.zeros((b_pad, _N_Q), jnp.float32).at[:B].set(
            Q.astype(jnp.float32))
    qT = Qp.T

    whole = lambda shp: pl.BlockSpec(shp, lambda i: (0, 0))
    out = pl.pallas_call(
        _fwd_kernel,
        out_shape=jax.ShapeDtypeStruct((_NS, b_pad), jnp.float32),
        grid=(grid,),
        in_specs=[
            pl.BlockSpec((_N_Q, _TILE), lambda i: (0, i)),
            whole((2 * _N_XI, _N_Q)), whole((2 * _N_XI, 1)),
            whole((_XR, 1)), whole((_XR, 1)),
            whole((_SR // 2, _XR // 2)), whole((_SR // 2, _SR // 2)),
            whole((_NS // 2, _SR // 2)), whole((_NS // 2, _SR // 2)),
            whole((_SR // 2, 1)), whole((_SR // 2, 1)), whole((1, 1)),
        ],
        out_specs=pl.BlockSpec((_NS, _TILE), lambda i: (0, i)),
        compiler_params=pltpu.CompilerParams(
            dimension_semantics=("parallel",)),
        cost_estimate=pl.CostEstimate(
            flops=_FLOPS_PER_CFG * b_pad,
            transcendentals=_TRANS_PER_CFG * b_pad,
            bytes_accessed=4 * (_N_Q + _NS) * b_pad + 4 * _SR * _SR),
    )(qT, wk2, offs, cph32, sph32, w14, w24, w3hi, w3lo, b1r, b2r, b3)
    return out[:, :B].T
